# baseline (speedup 1.0000x reference)
import functools

import jax
import jax.numpy as jnp
from jax import lax
from jax.experimental import pallas as pl
from jax.experimental.pallas import tpu as pltpu
from jax.experimental.pallas import tpu_sc as plsc

H = 32
D_MAX = 5
F = 3
KPP = D_MAX * F
E_ROWS = 1537
E_STRIDE = 1552
SP_ROWS = 512
SP_BASE = D_MAX * E_STRIDE
T_ROWS = SP_BASE + SP_ROWS

NC, NS = 2, 16
NW = NC * NS
C_PAIRS = 128
EPC = C_PAIRS * KPP
RPC = C_PAIRS * 16
NSTREAM = RPC // 128


def _table_body(ew_ref, dis_ref, spw_ref, out_ref):
    ew = ew_ref[...]
    dis2 = dis_ref[...].reshape(D_MAX * H, H)
    for d in range(D_MAX):
        w = jnp.dot(ew, dis2[d * H:(d + 1) * H, :],
                    preferred_element_type=jnp.float32) * (1.0 / 3.0)
        out_ref[pl.ds(d * E_STRIDE, E_ROWS), :] = w.astype(jnp.bfloat16)
    out_ref[pl.ds(SP_BASE, SP_ROWS), :] = spw_ref[...].astype(jnp.bfloat16)


def _build_table(ew, edw, spw):
    return pl.pallas_call(
        _table_body,
        grid=(1,),
        in_specs=[
            pl.BlockSpec(ew.shape, lambda i: (0, 0)),
            pl.BlockSpec((D_MAX * H * H, 1), lambda i: (0, 0)),
            pl.BlockSpec(spw.shape, lambda i: (0, 0)),
        ],
        out_specs=pl.BlockSpec((T_ROWS, H), lambda i: (0, 0)),
        out_shape=jax.ShapeDtypeStruct((T_ROWS, H), jnp.bfloat16),
    )(ew, edw, spw)


def _sc_gather_sum(table, edge1d, sp1d, p_total):
    ppw = p_total // NW
    nchunk = ppw // C_PAIRS
    nhalf = nchunk // 2
    mesh = plsc.VectorSubcoreMesh(core_axis_name="c", subcore_axis_name="s")

    @functools.partial(
        pl.kernel,
        out_type=jax.ShapeDtypeStruct((p_total, 2 * H), jnp.bfloat16),
        mesh=mesh,
        compiler_params=pltpu.CompilerParams(use_tc_tiling_on_sc=False),
        scratch_types=[
            pltpu.VMEM((2, 128), jnp.int32),
            pltpu.VMEM((EPC // 128, 128), jnp.int32),
            pltpu.VMEM((EPC // 128, 128), jnp.int32),
            pltpu.VMEM((1, 128), jnp.int32),
            pltpu.VMEM((1, 128), jnp.int32),
            pltpu.VMEM((NSTREAM, 128), jnp.int32),
            pltpu.VMEM((NSTREAM, 128), jnp.int32),
            pltpu.VMEM((RPC, H), jnp.bfloat16),
            pltpu.VMEM((RPC, H), jnp.bfloat16),
            pltpu.VMEM((C_PAIRS, 2 * H), jnp.bfloat16),
            pltpu.VMEM((C_PAIRS, 2 * H), jnp.bfloat16),
            pltpu.SemaphoreType.DMA,
            pltpu.SemaphoreType.DMA,
        ],
    )
    def k(table_hbm, edge_hbm, sp_hbm, doff_hbm, out_hbm,
          doff2, eraw0, eraw1, spraw0, spraw1, idx0, idx1,
          rows0, rows1, outb0, outb1, sem0, sem1):
        wid = lax.axis_index("s") * NC + lax.axis_index("c")
        pair_base = wid * ppw

        pltpu.sync_copy(doff_hbm, doff2)

        def load_and_index(ci, eraw, spraw, idx):
            erow0 = wid * (ppw * KPP // 128) + ci * (EPC // 128)
            srow0 = wid * (ppw // 128) + ci
            pltpu.sync_copy(edge_hbm.at[pl.ds(erow0, EPC // 128)], eraw)
            pltpu.sync_copy(sp_hbm.at[pl.ds(srow0, 1)], spraw)
            for v in range(EPC // 16):
                pat = v % KPP
                idx[v // 8, pl.ds((v % 8) * 16, 16)] = (
                    eraw[v // 8, pl.ds((v % 8) * 16, 16)]
                    + doff2[pat // 8, pl.ds((pat % 8) * 16, 16)])
            for w in range(C_PAIRS // 16):
                idx[NSTREAM - 1, pl.ds(w * 16, 16)] = (
                    spraw[0, pl.ds(w * 16, 16)] + SP_BASE)

        def fire(idx, rows, sem):
            for j in range(NSTREAM):
                pltpu.async_copy(table_hbm.at[idx.at[j]],
                                 rows.at[pl.ds(j * 128, 128)], sem)

        def drain(idx, rows, sem):
            for j in range(NSTREAM):
                pltpu.make_async_copy(
                    table_hbm.at[idx.at[j]],
                    rows.at[pl.ds(j * 128, 128)], sem).wait()

        def reduce_store(ci, rows, outb):
            def pair_body(p, c2):
                r0 = p * KPP
                v = [rows[r0 + t, 0:H] for t in range(KPP)]
                s1 = [v[2 * t] + v[2 * t + 1] for t in range(7)]
                s2 = [s1[2 * t] + s1[2 * t + 1] for t in range(3)]
                s3 = s2[0] + s2[1]
                outb[p, 0:H] = s3 + (s2[2] + v[14])
                outb[p, H:2 * H] = rows[EPC + p, 0:H]
                return c2

            lax.fori_loop(0, C_PAIRS, pair_body, 0)
            pair0 = pl.multiple_of(pair_base + ci * C_PAIRS, C_PAIRS)
            pltpu.sync_copy(outb, out_hbm.at[pl.ds(pair0, C_PAIRS)])

        load_and_index(0, eraw0, spraw0, idx0)
        fire(idx0, rows0, sem0)

        def body2(i, carry):
            c0 = i * 2
            load_and_index(c0 + 1, eraw1, spraw1, idx1)
            fire(idx1, rows1, sem1)
            drain(idx0, rows0, sem0)
            reduce_store(c0, rows0, outb0)

            @pl.when(i < nhalf - 1)
            def _():
                load_and_index(c0 + 2, eraw0, spraw0, idx0)
                fire(idx0, rows0, sem0)

            drain(idx1, rows1, sem1)
            reduce_store(c0 + 1, rows1, outb1)
            return carry

        lax.fori_loop(0, nhalf, body2, 0)

    doff_np = [((t % KPP) // F) * E_STRIDE for t in range(2 * 128)]
    doff_const = jnp.asarray(doff_np, dtype=jnp.int32).reshape(2, 128)
    return k(table, edge1d, sp1d, doff_const)


def _asm_body(ab_ref, int_ref, sp_ref, tok_ref, out_ref):
    x = int_ref[0]
    ii = lax.broadcasted_iota(jnp.int32, (H, H), 0)
    jj = lax.broadcasted_iota(jnp.int32, (H, H), 1)
    eye = (ii == jj).astype(jnp.bfloat16)
    dn = (((1,), (1,)), ((), ()))
    te = lax.dot_general(eye, x[:, 0:H], dn,
                         preferred_element_type=jnp.float32)
    ts = lax.dot_general(eye, x[:, H:2 * H], dn,
                         preferred_element_type=jnp.float32)
    n = ab_ref.shape[1] - 1
    sp = sp_ref[0]
    spc = jnp.clip(sp - 1, 1, 5)
    inv = 1.0 / spc.astype(jnp.float32)
    t = te.reshape(H, n, n) * inv[None, :, :] + ts.reshape(H, n, n)
    ab = ab_ref[0]
    tok = tok_ref[0, :]
    interior = t + 2.0 * ab[1:, 1:][None, :, :]
    col0 = 2.0 * ab[1:, 0][None, :] + tok[:, None]
    row0 = 2.0 * ab[0, :][None, :] + tok[:, None]
    body = jnp.concatenate([col0[:, :, None], interior], axis=2)
    out = jnp.concatenate([row0[:, None, :], body], axis=1)
    out_ref[0] = out


def _assemble(attn_bias, interior3, sp_nat, gtw):
    b, np1, _ = attn_bias.shape
    n = np1 - 1
    return pl.pallas_call(
        _asm_body,
        grid=(b,),
        in_specs=[
            pl.BlockSpec((1, np1, np1), lambda i: (i, 0, 0)),
            pl.BlockSpec((1, n * n, 2 * H), lambda i: (i, 0, 0)),
            pl.BlockSpec((1, n, n), lambda i: (i, 0, 0)),
            pl.BlockSpec((1, H), lambda i: (0, 0)),
        ],
        out_specs=pl.BlockSpec((1, H, np1, np1), lambda i: (i, 0, 0, 0)),
        out_shape=jax.ShapeDtypeStruct((b, H, np1, np1), jnp.float32),
    )(attn_bias, interior3, sp_nat, gtw)


def kernel(attn_bias, spatial_pos, x, edge_input, attn_edge_type,
           edge_encoder_w, edge_dis_encoder_w, spatial_pos_encoder_w,
           graph_token_w):
    b, np1, _ = attn_bias.shape
    n = np1 - 1
    p_total = b * n * n

    table = _build_table(edge_encoder_w, edge_dis_encoder_w,
                         spatial_pos_encoder_w)

    sp_nat = spatial_pos.astype(jnp.int32)
    edge2 = edge_input.astype(jnp.int32).reshape(p_total * KPP // 128, 128)
    sp2 = sp_nat.reshape(p_total // 128, 128)

    interior = _sc_gather_sum(table, edge2, sp2, p_total)
    return _assemble(attn_bias, interior.reshape(b, n * n, 2 * H),
                     sp_nat, graph_token_w)


# speedup vs baseline: 3.4497x; 3.4497x over previous
import functools

import jax
import jax.numpy as jnp
from jax import lax
from jax.experimental import pallas as pl
from jax.experimental.pallas import tpu as pltpu
from jax.experimental.pallas import tpu_sc as plsc

H = 32
D_MAX = 5
F = 3
KPP = D_MAX * F
E_ROWS = 1537
E_STRIDE = 1552
SP_ROWS = 512
SP_BASE = D_MAX * E_STRIDE
T_ROWS = SP_BASE + SP_ROWS

NC, NS = 2, 16
NW = NC * NS
C_PAIRS = 128
EPC = C_PAIRS * KPP
RPC = C_PAIRS * 16
NSTREAM = RPC // 128


def _table_body(ew_ref, dis_ref, spw_ref, out_ref):
    ew = ew_ref[...]
    dis2 = dis_ref[...].reshape(D_MAX * H, H)
    for d in range(D_MAX):
        w = jnp.dot(ew, dis2[d * H:(d + 1) * H, :],
                    preferred_element_type=jnp.float32) * (1.0 / 3.0)
        out_ref[pl.ds(d * E_STRIDE, E_ROWS), :] = w.astype(jnp.bfloat16)
    out_ref[pl.ds(SP_BASE, SP_ROWS), :] = spw_ref[...].astype(jnp.bfloat16)


def _build_table(ew, edw, spw):
    return pl.pallas_call(
        _table_body,
        grid=(1,),
        in_specs=[
            pl.BlockSpec(ew.shape, lambda i: (0, 0)),
            pl.BlockSpec((D_MAX * H * H, 1), lambda i: (0, 0)),
            pl.BlockSpec(spw.shape, lambda i: (0, 0)),
        ],
        out_specs=pl.BlockSpec((T_ROWS, H), lambda i: (0, 0)),
        out_shape=jax.ShapeDtypeStruct((T_ROWS, H), jnp.bfloat16),
    )(ew, edw, spw)


def _sc_gather_sum(table, edge1d, sp1d, p_total):
    ppw = p_total // NW
    nchunk = ppw // C_PAIRS
    nhalf = nchunk // 2
    mesh = plsc.VectorSubcoreMesh(core_axis_name="c", subcore_axis_name="s")

    @functools.partial(
        pl.kernel,
        out_type=jax.ShapeDtypeStruct((p_total, 2 * H), jnp.bfloat16),
        mesh=mesh,
        compiler_params=pltpu.CompilerParams(use_tc_tiling_on_sc=False),
        scratch_types=[
            pltpu.VMEM((KPP, 128), jnp.int32),
            pltpu.VMEM((KPP, 128), jnp.int32),
            pltpu.VMEM((1, 128), jnp.int32),
            pltpu.VMEM((1, 128), jnp.int32),
            pltpu.VMEM((NSTREAM, 128), jnp.int32),
            pltpu.VMEM((NSTREAM, 128), jnp.int32),
            pltpu.VMEM((RPC, H), jnp.bfloat16),
            pltpu.VMEM((RPC, H), jnp.bfloat16),
            pltpu.VMEM((C_PAIRS, 2 * H), jnp.bfloat16),
            pltpu.VMEM((C_PAIRS, 2 * H), jnp.bfloat16),
            pltpu.SemaphoreType.DMA,
            pltpu.SemaphoreType.DMA,
        ],
    )
    def k(table_hbm, edge_hbm, sp_hbm, out_hbm,
          eraw0, eraw1, spraw0, spraw1, idx0, idx1,
          rows0, rows1, outb0, outb1, sem0, sem1):
        wid = lax.axis_index("s") * NC + lax.axis_index("c")
        pair_base = wid * ppw

        def load_and_index(ci, eraw, spraw, idx):
            pair0 = pair_base + ci * C_PAIRS
            b = pair0 // 16384
            rr = (pair0 // 128) % 128
            srow0 = pair0 // 128
            pltpu.sync_copy(edge_hbm.at[pl.ds(b * KPP, KPP), rr], eraw)
            pltpu.sync_copy(sp_hbm.at[pl.ds(srow0, 1)], spraw)
            for kk in range(KPP):
                for v in range(8):
                    idx[kk, pl.ds(v * 16, 16)] = (
                        eraw[kk, pl.ds(v * 16, 16)] + (kk // F) * E_STRIDE)
            for w in range(C_PAIRS // 16):
                idx[NSTREAM - 1, pl.ds(w * 16, 16)] = (
                    spraw[0, pl.ds(w * 16, 16)] + SP_BASE)

        def fire(idx, rows, sem):
            for j in range(NSTREAM):
                pltpu.async_copy(table_hbm.at[idx.at[j]],
                                 rows.at[pl.ds(j * 128, 128)], sem)

        def drain(idx, rows, sem):
            for j in range(NSTREAM):
                pltpu.make_async_copy(
                    table_hbm.at[idx.at[j]],
                    rows.at[pl.ds(j * 128, 128)], sem).wait()

        def reduce_store(ci, rows, outb):
            def pair_body(p, c2):
                v = [rows[t * C_PAIRS + p, 0:H] for t in range(KPP)]
                s1 = [v[2 * t] + v[2 * t + 1] for t in range(7)]
                s2 = [s1[2 * t] + s1[2 * t + 1] for t in range(3)]
                s3 = s2[0] + s2[1]
                outb[p, 0:H] = s3 + (s2[2] + v[14])
                outb[p, H:2 * H] = rows[EPC + p, 0:H]
                return c2

            lax.fori_loop(0, C_PAIRS, pair_body, 0)
            pair0 = pl.multiple_of(pair_base + ci * C_PAIRS, C_PAIRS)
            pltpu.sync_copy(outb, out_hbm.at[pl.ds(pair0, C_PAIRS)])

        load_and_index(0, eraw0, spraw0, idx0)
        fire(idx0, rows0, sem0)

        def body2(i, carry):
            c0 = i * 2
            load_and_index(c0 + 1, eraw1, spraw1, idx1)
            fire(idx1, rows1, sem1)
            drain(idx0, rows0, sem0)
            reduce_store(c0, rows0, outb0)

            @pl.when(i < nhalf - 1)
            def _():
                load_and_index(c0 + 2, eraw0, spraw0, idx0)
                fire(idx0, rows0, sem0)

            drain(idx1, rows1, sem1)
            reduce_store(c0 + 1, rows1, outb1)
            return carry

        lax.fori_loop(0, nhalf, body2, 0)

    return k(table, edge1d, sp1d)


def _asm_body(ab_ref, int_ref, sp_ref, tok_ref, out_ref):
    x = int_ref[0]
    ii = lax.broadcasted_iota(jnp.int32, (H, H), 0)
    jj = lax.broadcasted_iota(jnp.int32, (H, H), 1)
    eye = (ii == jj).astype(jnp.bfloat16)
    dn = (((1,), (1,)), ((), ()))
    te = lax.dot_general(eye, x[:, 0:H], dn,
                         preferred_element_type=jnp.float32)
    ts = lax.dot_general(eye, x[:, H:2 * H], dn,
                         preferred_element_type=jnp.float32)
    n = ab_ref.shape[1] - 1
    sp = sp_ref[0]
    spc = jnp.clip(sp - 1, 1, 5)
    inv = 1.0 / spc.astype(jnp.float32)
    t = te.reshape(H, n, n) * inv[None, :, :] + ts.reshape(H, n, n)
    ab = ab_ref[0]
    tok = tok_ref[0, :]
    interior = t + 2.0 * ab[1:, 1:][None, :, :]
    col0 = 2.0 * ab[1:, 0][None, :] + tok[:, None]
    row0 = 2.0 * ab[0, :][None, :] + tok[:, None]
    body = jnp.concatenate([col0[:, :, None], interior], axis=2)
    out = jnp.concatenate([row0[:, None, :], body], axis=1)
    out_ref[0] = out


def _assemble(attn_bias, interior3, sp_nat, gtw):
    b, np1, _ = attn_bias.shape
    n = np1 - 1
    return pl.pallas_call(
        _asm_body,
        grid=(b,),
        in_specs=[
            pl.BlockSpec((1, np1, np1), lambda i: (i, 0, 0)),
            pl.BlockSpec((1, n * n, 2 * H), lambda i: (i, 0, 0)),
            pl.BlockSpec((1, n, n), lambda i: (i, 0, 0)),
            pl.BlockSpec((1, H), lambda i: (0, 0)),
        ],
        out_specs=pl.BlockSpec((1, H, np1, np1), lambda i: (i, 0, 0, 0)),
        out_shape=jax.ShapeDtypeStruct((b, H, np1, np1), jnp.float32),
    )(attn_bias, interior3, sp_nat, gtw)


def kernel(attn_bias, spatial_pos, x, edge_input, attn_edge_type,
           edge_encoder_w, edge_dis_encoder_w, spatial_pos_encoder_w,
           graph_token_w):
    b, np1, _ = attn_bias.shape
    n = np1 - 1
    p_total = b * n * n

    table = _build_table(edge_encoder_w, edge_dis_encoder_w,
                         spatial_pos_encoder_w)

    sp_nat = spatial_pos.astype(jnp.int32)
    edge_k = jnp.transpose(edge_input.astype(jnp.int32),
                           (0, 3, 4, 1, 2)).reshape(b * KPP, n, n)
    sp2 = sp_nat.reshape(p_total // 128, 128)

    interior = _sc_gather_sum(table, edge_k, sp2, p_total)
    return _assemble(attn_bias, interior.reshape(b, n * n, 2 * H),
                     sp_nat, graph_token_w)
